# Initial kernel scaffold; baseline (speedup 1.0000x reference)
import jax
import jax.numpy as jnp
from jax.experimental import pallas as pl


def kernel(x, edge_index, W1, b1, W2, b2, W3, b3):
    raise NotImplementedError("write your pallas kernel here")


# speedup vs baseline: 6.5578x; 6.5578x over previous
import functools

import jax
import jax.numpy as jnp
from jax import lax
from jax.experimental import pallas as pl
from jax.experimental.pallas import tpu as pltpu
from jax.experimental.pallas import tpu_sc as plsc

N = 10000
D = 128
E = 320000

N_PAD = 10240
E_PAD = 327680
NW = 32
EDGES_PER_TILE = E_PAD // NW
CHUNK = 128
N_CHUNKS = EDGES_PER_TILE // CHUNK
SUB_ROWS = N_PAD // 16
DEG_W = 128
DUMP_ROW = N
BLK = 1024

_MESH = plsc.VectorSubcoreMesh(core_axis_name="c", subcore_axis_name="s")


def _sc_degree(dst3, ones_hbm, zeros16):
    @functools.partial(
        pl.kernel,
        out_type=jax.ShapeDtypeStruct((2, N_PAD, DEG_W), jnp.float32),
        mesh=_MESH,
        scratch_types=[
            pltpu.VMEM((N_CHUNKS, CHUNK), jnp.int32),
            pltpu.VMEM((CHUNK, DEG_W), jnp.float32),
            pltpu.VMEM_SHARED((N_PAD, DEG_W), jnp.float32),
        ],
    )
    def k(dst_hbm, ones_h, zeros_h, out_hbm, idx_v, ones_v, hist_sh):
        c = lax.axis_index("c")
        s = lax.axis_index("s")
        wid = s * 2 + c
        pltpu.sync_copy(dst_hbm.at[wid], idx_v)
        pltpu.sync_copy(ones_h, ones_v)
        rows = pl.ds(s * SUB_ROWS, SUB_ROWS)
        pltpu.sync_copy(zeros_h.at[rows], hist_sh.at[rows])
        plsc.subcore_barrier()

        @pl.loop(0, N_CHUNKS)
        def _(j):
            pltpu.sync_copy(ones_v, hist_sh.at[idx_v.at[j]], add=True)

        plsc.subcore_barrier()
        pltpu.sync_copy(hist_sh.at[rows], out_hbm.at[c].at[rows])

    return k(dst3, ones_hbm, zeros16)


def _sc_segsum(g, src3, dst3, zeros_nd):
    @functools.partial(
        pl.kernel,
        out_type=jax.ShapeDtypeStruct((2, N_PAD, D), jnp.float32),
        mesh=_MESH,
        scratch_types=[
            pltpu.VMEM((N_CHUNKS, CHUNK), jnp.int32),
            pltpu.VMEM((N_CHUNKS, CHUNK), jnp.int32),
            pltpu.VMEM((CHUNK, D), jnp.float32),
            pltpu.VMEM_SHARED((N_PAD, D), jnp.float32),
            pltpu.SemaphoreType.DMA,
        ],
    )
    def k(g_hbm, src_hbm, dst_hbm, zeros_h, out_hbm, src_v, dst_v, buf, agg_sh, sem):
        c = lax.axis_index("c")
        s = lax.axis_index("s")
        wid = s * 2 + c
        pltpu.sync_copy(src_hbm.at[wid], src_v)
        pltpu.sync_copy(dst_hbm.at[wid], dst_v)
        rows = pl.ds(s * SUB_ROWS, SUB_ROWS)

        @pl.when(c == 0)
        def _():
            pltpu.sync_copy(g_hbm.at[rows], agg_sh.at[rows])

        @pl.when(c == 1)
        def _():
            pltpu.sync_copy(zeros_h.at[rows], agg_sh.at[rows])

        plsc.subcore_barrier()

        @pl.loop(0, N_CHUNKS)
        def _(j):
            pltpu.async_copy(g_hbm.at[src_v.at[j]], buf, sem).wait()
            pltpu.sync_copy(buf, agg_sh.at[dst_v.at[j]], add=True)

        plsc.subcore_barrier()
        pltpu.sync_copy(agg_sh.at[rows], out_hbm.at[c].at[rows])

    return k(g, src3, dst3, zeros_nd)


def _tc_prep(deg_part, x_pad, W1):
    def body(deg_ref, x_ref, w_ref, g_ref, dinv_ref):
        deg = deg_ref[0, :, 0:1] + deg_ref[1, :, 0:1] + 1.0
        dinv = lax.rsqrt(deg)
        h = jnp.dot(x_ref[...], w_ref[...], preferred_element_type=jnp.float32)
        g_ref[...] = h * dinv
        dinv_ref[...] = dinv

    return pl.pallas_call(
        body,
        grid=(N_PAD // BLK,),
        in_specs=[
            pl.BlockSpec((2, BLK, DEG_W), lambda i: (0, i, 0)),
            pl.BlockSpec((BLK, D), lambda i: (i, 0)),
            pl.BlockSpec((D, D), lambda i: (0, 0)),
        ],
        out_specs=[
            pl.BlockSpec((BLK, D), lambda i: (i, 0)),
            pl.BlockSpec((BLK, 1), lambda i: (i, 0)),
        ],
        out_shape=[
            jax.ShapeDtypeStruct((N_PAD, D), jnp.float32),
            jax.ShapeDtypeStruct((N_PAD, 1), jnp.float32),
        ],
    )(deg_part, x_pad, W1)


def _tc_layer(p, dinv, b2d, W_next):
    def body(p_ref, dinv_ref, b_ref, w_ref, out_ref):
        t = (p_ref[0] + p_ref[1]) * dinv_ref[...] + b_ref[...]
        t = jnp.maximum(t, 0.0)
        out_ref[...] = (
            jnp.dot(t, w_ref[...], preferred_element_type=jnp.float32)
            * dinv_ref[...]
        )

    return pl.pallas_call(
        body,
        grid=(N_PAD // BLK,),
        in_specs=[
            pl.BlockSpec((2, BLK, D), lambda i: (0, i, 0)),
            pl.BlockSpec((BLK, 1), lambda i: (i, 0)),
            pl.BlockSpec((1, D), lambda i: (0, 0)),
            pl.BlockSpec((D, D), lambda i: (0, 0)),
        ],
        out_specs=pl.BlockSpec((BLK, D), lambda i: (i, 0)),
        out_shape=jax.ShapeDtypeStruct((N_PAD, D), jnp.float32),
    )(p, dinv, b2d, W_next)


def _tc_final(p, dinv, b2d):
    def body(p_ref, dinv_ref, b_ref, out_ref):
        out_ref[...] = (p_ref[0] + p_ref[1]) * dinv_ref[...] + b_ref[...]

    return pl.pallas_call(
        body,
        grid=(N_PAD // BLK,),
        in_specs=[
            pl.BlockSpec((2, BLK, D), lambda i: (0, i, 0)),
            pl.BlockSpec((BLK, 1), lambda i: (i, 0)),
            pl.BlockSpec((1, D), lambda i: (0, 0)),
        ],
        out_specs=pl.BlockSpec((BLK, D), lambda i: (i, 0)),
        out_shape=jax.ShapeDtypeStruct((N_PAD, D), jnp.float32),
    )(p, dinv, b2d)


def kernel(x, edge_index, W1, b1, W2, b2, W3, b3):
    src = edge_index[0]
    dst = edge_index[1]
    pad_e = E_PAD - E
    src3 = jnp.concatenate(
        [src, jnp.zeros((pad_e,), jnp.int32)]
    ).reshape(NW, N_CHUNKS, CHUNK)
    dst3 = jnp.concatenate(
        [dst, jnp.full((pad_e,), DUMP_ROW, jnp.int32)]
    ).reshape(NW, N_CHUNKS, CHUNK)
    x_pad = jnp.pad(x, ((0, N_PAD - N), (0, 0)))
    zeros_nd = jnp.zeros((N_PAD, D), jnp.float32)
    ones_hbm = jnp.ones((CHUNK, DEG_W), jnp.float32)

    deg_part = _sc_degree(dst3, ones_hbm, zeros_nd)
    g1, dinv = _tc_prep(deg_part, x_pad, W1)
    p1 = _sc_segsum(g1, src3, dst3, zeros_nd)
    g2 = _tc_layer(p1, dinv, b1.reshape(1, D), W2)
    p2 = _sc_segsum(g2, src3, dst3, zeros_nd)
    g3 = _tc_layer(p2, dinv, b2.reshape(1, D), W3)
    p3 = _sc_segsum(g3, src3, dst3, zeros_nd)
    return _tc_final(p3, dinv, b3.reshape(1, D))[:N]


# speedup vs baseline: 7.2155x; 1.1003x over previous
import dataclasses
import functools

import jax
import jax.numpy as jnp
from jax import lax
from jax.experimental import pallas as pl
from jax.experimental.pallas import tpu as pltpu
from jax.experimental.pallas import tpu_sc as plsc

N = 10000
D = 128
E = 320000

N_PAD = 10240
E_PAD = 327680
NW = 32
EDGES_PER_TILE = E_PAD // NW
CHUNK = 128
N_CHUNKS = EDGES_PER_TILE // CHUNK
SUB_ROWS = N_PAD // 16
DUMP_ROW = N
BLK = 1024

_MESH = plsc.VectorSubcoreMesh(core_axis_name="c", subcore_axis_name="s")

_CP = pltpu.CompilerParams()
if "needs_layout_passes" in pltpu.CompilerParams.__dataclass_fields__:
    _CP = dataclasses.replace(_CP, needs_layout_passes=False)


def _sc_degree(dst3):
    @functools.partial(
        pl.kernel,
        out_type=jax.ShapeDtypeStruct((NW, N_PAD), jnp.float32),
        mesh=_MESH,
        compiler_params=_CP,
        scratch_types=[
            pltpu.VMEM((N_CHUNKS, CHUNK), jnp.int32),
            pltpu.VMEM((N_PAD,), jnp.float32),
        ],
    )
    def k(dst_hbm, out_hbm, idx_v, hist_v):
        c = lax.axis_index("c")
        s = lax.axis_index("s")
        wid = s * 2 + c
        pltpu.sync_copy(dst_hbm.at[wid], idx_v)

        @pl.loop(0, N_PAD // 16)
        def _(i):
            hist_v[pl.ds(i * 16, 16)] = jnp.zeros((16,), jnp.float32)

        ones = jnp.ones((16,), jnp.float32)

        @pl.loop(0, N_CHUNKS)
        def _(j):
            @pl.loop(0, CHUNK // 16)
            def _(t):
                plsc.addupdate_scatter(
                    hist_v, [idx_v[j, pl.ds(t * 16, 16)]], ones
                )

        pltpu.sync_copy(hist_v, out_hbm.at[wid])

    return k(dst3)


def _tc_dinv(deg_part):
    def body(p_ref, out_ref):
        deg = jnp.sum(p_ref[...], axis=0, keepdims=True) + 1.0
        out_ref[...] = lax.rsqrt(deg)

    return pl.pallas_call(
        body,
        out_shape=jax.ShapeDtypeStruct((1, N_PAD), jnp.float32),
    )(deg_part)


def _sc_segsum(g, src3, dst3):
    @functools.partial(
        pl.kernel,
        out_type=jax.ShapeDtypeStruct((2, N_PAD, D), jnp.float32),
        mesh=_MESH,
        scratch_types=[
            pltpu.VMEM((N_CHUNKS, CHUNK), jnp.int32),
            pltpu.VMEM((N_CHUNKS, CHUNK), jnp.int32),
            pltpu.VMEM((CHUNK, D), jnp.float32),
            pltpu.VMEM_SHARED((N_PAD, D), jnp.float32),
            pltpu.SemaphoreType.DMA,
        ],
    )
    def k(g_hbm, src_hbm, dst_hbm, out_hbm, src_v, dst_v, buf, agg_sh, sem):
        c = lax.axis_index("c")
        s = lax.axis_index("s")
        wid = s * 2 + c
        pltpu.sync_copy(src_hbm.at[wid], src_v)
        pltpu.sync_copy(dst_hbm.at[wid], dst_v)
        rows = pl.ds(s * SUB_ROWS, SUB_ROWS)
        pltpu.sync_copy(g_hbm.at[rows], agg_sh.at[rows])
        plsc.subcore_barrier()

        @pl.loop(0, N_CHUNKS)
        def _(j):
            pltpu.async_copy(g_hbm.at[src_v.at[j]], buf, sem).wait()
            pltpu.sync_copy(buf, agg_sh.at[dst_v.at[j]], add=True)

        plsc.subcore_barrier()
        pltpu.sync_copy(agg_sh.at[rows], out_hbm.at[c].at[rows])

    return k(g, src3, dst3)


def _tc_prep(dinv, x_pad, W1):
    def body(dinv_ref, x_ref, w_ref, g_ref):
        h = jnp.dot(x_ref[...], w_ref[...], preferred_element_type=jnp.float32)
        g_ref[...] = h * dinv_ref[...]

    return pl.pallas_call(
        body,
        grid=(N_PAD // BLK,),
        in_specs=[
            pl.BlockSpec((BLK, 1), lambda i: (i, 0)),
            pl.BlockSpec((BLK, D), lambda i: (i, 0)),
            pl.BlockSpec((D, D), lambda i: (0, 0)),
        ],
        out_specs=pl.BlockSpec((BLK, D), lambda i: (i, 0)),
        out_shape=jax.ShapeDtypeStruct((N_PAD, D), jnp.float32),
    )(dinv, x_pad, W1)


def _tc_layer(p, g, dinv, b2d, W_next):
    def body(p_ref, g_ref, dinv_ref, b_ref, w_ref, out_ref):
        t = (p_ref[0] + p_ref[1] - g_ref[...]) * dinv_ref[...] + b_ref[...]
        t = jnp.maximum(t, 0.0)
        out_ref[...] = (
            jnp.dot(t, w_ref[...], preferred_element_type=jnp.float32)
            * dinv_ref[...]
        )

    return pl.pallas_call(
        body,
        grid=(N_PAD // BLK,),
        in_specs=[
            pl.BlockSpec((2, BLK, D), lambda i: (0, i, 0)),
            pl.BlockSpec((BLK, D), lambda i: (i, 0)),
            pl.BlockSpec((BLK, 1), lambda i: (i, 0)),
            pl.BlockSpec((1, D), lambda i: (0, 0)),
            pl.BlockSpec((D, D), lambda i: (0, 0)),
        ],
        out_specs=pl.BlockSpec((BLK, D), lambda i: (i, 0)),
        out_shape=jax.ShapeDtypeStruct((N_PAD, D), jnp.float32),
    )(p, g, dinv, b2d, W_next)


def _tc_final(p, g, dinv, b2d):
    def body(p_ref, g_ref, dinv_ref, b_ref, out_ref):
        out_ref[...] = (
            (p_ref[0] + p_ref[1] - g_ref[...]) * dinv_ref[...] + b_ref[...]
        )

    return pl.pallas_call(
        body,
        grid=(N_PAD // BLK,),
        in_specs=[
            pl.BlockSpec((2, BLK, D), lambda i: (0, i, 0)),
            pl.BlockSpec((BLK, D), lambda i: (i, 0)),
            pl.BlockSpec((BLK, 1), lambda i: (i, 0)),
            pl.BlockSpec((1, D), lambda i: (0, 0)),
        ],
        out_specs=pl.BlockSpec((BLK, D), lambda i: (i, 0)),
        out_shape=jax.ShapeDtypeStruct((N_PAD, D), jnp.float32),
    )(p, g, dinv, b2d)


def kernel(x, edge_index, W1, b1, W2, b2, W3, b3):
    src = edge_index[0]
    dst = edge_index[1]
    pad_e = E_PAD - E
    src3 = jnp.concatenate(
        [src, jnp.zeros((pad_e,), jnp.int32)]
    ).reshape(NW, N_CHUNKS, CHUNK)
    dst3 = jnp.concatenate(
        [dst, jnp.full((pad_e,), DUMP_ROW, jnp.int32)]
    ).reshape(NW, N_CHUNKS, CHUNK)
    x_pad = jnp.pad(x, ((0, N_PAD - N), (0, 0)))

    deg_part = _sc_degree(dst3)
    dinv = _tc_dinv(deg_part).reshape(N_PAD, 1)
    g1 = _tc_prep(dinv, x_pad, W1)
    p1 = _sc_segsum(g1, src3, dst3)
    g2 = _tc_layer(p1, g1, dinv, b1.reshape(1, D), W2)
    p2 = _sc_segsum(g2, src3, dst3)
    g3 = _tc_layer(p2, g2, dinv, b2.reshape(1, D), W3)
    p3 = _sc_segsum(g3, src3, dst3)
    return _tc_final(p3, g3, dinv, b3.reshape(1, D))[:N]


# speedup vs baseline: 8.3351x; 1.1552x over previous
import dataclasses
import functools

import jax
import jax.numpy as jnp
from jax import lax
from jax.experimental import pallas as pl
from jax.experimental.pallas import tpu as pltpu
from jax.experimental.pallas import tpu_sc as plsc

N = 10000
D = 128
E = 320000

N_PAD = 10240
E_PAD = 327680
NW = 32
EDGES_PER_TILE = E_PAD // NW
CHUNK = 64
N_CHUNKS = EDGES_PER_TILE // CHUNK
NG = 16
NGROUPS = N_CHUNKS // NG
SUB_ROWS = N_PAD // 16
DUMP_ROW = N
BLK = 1024

_MESH = plsc.VectorSubcoreMesh(core_axis_name="c", subcore_axis_name="s")

_CP = pltpu.CompilerParams()
if "needs_layout_passes" in pltpu.CompilerParams.__dataclass_fields__:
    _CP = dataclasses.replace(_CP, needs_layout_passes=False)


def _sc_degree(dst3):
    @functools.partial(
        pl.kernel,
        out_type=jax.ShapeDtypeStruct((NW, N_PAD), jnp.float32),
        mesh=_MESH,
        compiler_params=_CP,
        scratch_types=[
            pltpu.VMEM((N_CHUNKS, CHUNK), jnp.int32),
            pltpu.VMEM((N_PAD,), jnp.float32),
        ],
    )
    def k(dst_hbm, out_hbm, idx_v, hist_v):
        c = lax.axis_index("c")
        s = lax.axis_index("s")
        wid = s * 2 + c
        pltpu.sync_copy(dst_hbm.at[wid], idx_v)

        @pl.loop(0, N_PAD // 16)
        def _(i):
            hist_v[pl.ds(i * 16, 16)] = jnp.zeros((16,), jnp.float32)

        ones = jnp.ones((16,), jnp.float32)

        @pl.loop(0, N_CHUNKS)
        def _(j):
            @pl.loop(0, CHUNK // 16)
            def _(t):
                plsc.addupdate_scatter(
                    hist_v, [idx_v[j, pl.ds(t * 16, 16)]], ones
                )

        pltpu.sync_copy(hist_v, out_hbm.at[wid])

    return k(dst3)


def _tc_dinv(deg_part):
    def body(p_ref, out_ref):
        deg = jnp.sum(p_ref[...], axis=0, keepdims=True) + 1.0
        out_ref[...] = lax.rsqrt(deg)

    return pl.pallas_call(
        body,
        out_shape=jax.ShapeDtypeStruct((1, N_PAD), jnp.float32),
    )(deg_part)


def _sc_segsum(g, src3, dst3):
    @functools.partial(
        pl.kernel,
        out_type=jax.ShapeDtypeStruct((2, N_PAD, D), jnp.float32),
        mesh=_MESH,
        scratch_types=[
            pltpu.VMEM((2, NG, CHUNK), jnp.int32),
            pltpu.VMEM((2, NG, CHUNK), jnp.int32),
            pltpu.VMEM((2 * CHUNK, D), jnp.float32),
            pltpu.VMEM_SHARED((N_PAD, D), jnp.float32),
            pltpu.SemaphoreType.DMA,
            pltpu.SemaphoreType.DMA,
            pltpu.SemaphoreType.DMA,
            pltpu.SemaphoreType.DMA,
        ],
    )
    def k(g_hbm, src_hbm, dst_hbm, out_hbm, si, di, buf, agg_sh,
          sg0, sg1, is0, is1):
        c = lax.axis_index("c")
        s = lax.axis_index("s")
        wid = s * 2 + c
        rows = pl.ds(s * SUB_ROWS, SUB_ROWS)
        halves = (buf.at[pl.ds(0, CHUNK)], buf.at[pl.ds(CHUNK, CHUNK)])
        sgs = (sg0, sg1)
        iss = (is0, is1)

        def idx_start(q, gg):
            pltpu.async_copy(src_hbm.at[wid, pl.ds(gg * NG, NG)], si.at[q], iss[q])
            pltpu.async_copy(dst_hbm.at[wid, pl.ds(gg * NG, NG)], di.at[q], iss[q])

        def idx_wait(q, gg):
            pltpu.make_async_copy(
                src_hbm.at[wid, pl.ds(gg * NG, NG)], si.at[q], iss[q]
            ).wait()
            pltpu.make_async_copy(
                dst_hbm.at[wid, pl.ds(gg * NG, NG)], di.at[q], iss[q]
            ).wait()

        for q in range(2):
            idx_start(q, q)
        pltpu.sync_copy(g_hbm.at[rows], agg_sh.at[rows])
        plsc.subcore_barrier()

        @pl.loop(0, NGROUPS, step=2)
        def _(g):
            for q in range(2):
                gg = g + q
                idx_wait(q, gg)
                for b in range(2):
                    pltpu.async_copy(g_hbm.at[si.at[q, b]], halves[b], sgs[b])
                for t in range(NG):
                    b = t % 2
                    pltpu.make_async_copy(
                        g_hbm.at[si.at[q, t]], halves[b], sgs[b]
                    ).wait()
                    pltpu.sync_copy(halves[b], agg_sh.at[di.at[q, t]], add=True)
                    if t + 2 < NG:
                        pltpu.async_copy(g_hbm.at[si.at[q, t + 2]], halves[b], sgs[b])

                @pl.when(gg + 2 < NGROUPS)
                def _(q=q, gg=gg):
                    idx_start(q, gg + 2)

        plsc.subcore_barrier()
        pltpu.sync_copy(agg_sh.at[rows], out_hbm.at[c].at[rows])

    return k(g, src3, dst3)


def _tc_prep(dinv, x_pad, W1):
    def body(dinv_ref, x_ref, w_ref, g_ref):
        h = jnp.dot(x_ref[...], w_ref[...], preferred_element_type=jnp.float32)
        g_ref[...] = h * dinv_ref[...]

    return pl.pallas_call(
        body,
        grid=(N_PAD // BLK,),
        in_specs=[
            pl.BlockSpec((BLK, 1), lambda i: (i, 0)),
            pl.BlockSpec((BLK, D), lambda i: (i, 0)),
            pl.BlockSpec((D, D), lambda i: (0, 0)),
        ],
        out_specs=pl.BlockSpec((BLK, D), lambda i: (i, 0)),
        out_shape=jax.ShapeDtypeStruct((N_PAD, D), jnp.float32),
    )(dinv, x_pad, W1)


def _tc_layer(p, g, dinv, b2d, W_next):
    def body(p_ref, g_ref, dinv_ref, b_ref, w_ref, out_ref):
        t = (p_ref[0] + p_ref[1] - g_ref[...]) * dinv_ref[...] + b_ref[...]
        t = jnp.maximum(t, 0.0)
        out_ref[...] = (
            jnp.dot(t, w_ref[...], preferred_element_type=jnp.float32)
            * dinv_ref[...]
        )

    return pl.pallas_call(
        body,
        grid=(N_PAD // BLK,),
        in_specs=[
            pl.BlockSpec((2, BLK, D), lambda i: (0, i, 0)),
            pl.BlockSpec((BLK, D), lambda i: (i, 0)),
            pl.BlockSpec((BLK, 1), lambda i: (i, 0)),
            pl.BlockSpec((1, D), lambda i: (0, 0)),
            pl.BlockSpec((D, D), lambda i: (0, 0)),
        ],
        out_specs=pl.BlockSpec((BLK, D), lambda i: (i, 0)),
        out_shape=jax.ShapeDtypeStruct((N_PAD, D), jnp.float32),
    )(p, g, dinv, b2d, W_next)


def _tc_final(p, g, dinv, b2d):
    def body(p_ref, g_ref, dinv_ref, b_ref, out_ref):
        out_ref[...] = (
            (p_ref[0] + p_ref[1] - g_ref[...]) * dinv_ref[...] + b_ref[...]
        )

    return pl.pallas_call(
        body,
        grid=(N_PAD // BLK,),
        in_specs=[
            pl.BlockSpec((2, BLK, D), lambda i: (0, i, 0)),
            pl.BlockSpec((BLK, D), lambda i: (i, 0)),
            pl.BlockSpec((BLK, 1), lambda i: (i, 0)),
            pl.BlockSpec((1, D), lambda i: (0, 0)),
        ],
        out_specs=pl.BlockSpec((BLK, D), lambda i: (i, 0)),
        out_shape=jax.ShapeDtypeStruct((N_PAD, D), jnp.float32),
    )(p, g, dinv, b2d)


def kernel(x, edge_index, W1, b1, W2, b2, W3, b3):
    src = edge_index[0]
    dst = edge_index[1]
    pad_e = E_PAD - E
    src3 = jnp.concatenate(
        [src, jnp.zeros((pad_e,), jnp.int32)]
    ).reshape(NW, N_CHUNKS, CHUNK)
    dst3 = jnp.concatenate(
        [dst, jnp.full((pad_e,), DUMP_ROW, jnp.int32)]
    ).reshape(NW, N_CHUNKS, CHUNK)
    x_pad = jnp.pad(x, ((0, N_PAD - N), (0, 0)))

    deg_part = _sc_degree(dst3)
    dinv = _tc_dinv(deg_part).reshape(N_PAD, 1)
    g1 = _tc_prep(dinv, x_pad, W1)
    p1 = _sc_segsum(g1, src3, dst3)
    g2 = _tc_layer(p1, g1, dinv, b1.reshape(1, D), W2)
    p2 = _sc_segsum(g2, src3, dst3)
    g3 = _tc_layer(p2, g2, dinv, b2.reshape(1, D), W3)
    p3 = _sc_segsum(g3, src3, dst3)
    return _tc_final(p3, g3, dinv, b3.reshape(1, D))[:N]


# speedup vs baseline: 23.7399x; 2.8482x over previous
import dataclasses
import functools

import jax
import jax.numpy as jnp
from jax import lax
from jax.experimental import pallas as pl
from jax.experimental.pallas import tpu as pltpu
from jax.experimental.pallas import tpu_sc as plsc

N = 10000
D = 128
E = 320000

N_PAD = 10240
E_PAD = 327680
NW = 32
EDGES_PER_TILE = E_PAD // NW
CHUNK = 64
N_CHUNKS = EDGES_PER_TILE // CHUNK
NG = 16
NGROUPS = N_CHUNKS // NG
SUB_ROWS = N_PAD // 16
DUMP_ROW = N
BLK = 1024

_MESH = plsc.VectorSubcoreMesh(core_axis_name="c", subcore_axis_name="s")

_CP = pltpu.CompilerParams()
if "needs_layout_passes" in pltpu.CompilerParams.__dataclass_fields__:
    _CP = dataclasses.replace(_CP, needs_layout_passes=False)


def _sc_degree(dst3):
    @functools.partial(
        pl.kernel,
        out_type=jax.ShapeDtypeStruct((NW, N_PAD), jnp.float32),
        mesh=_MESH,
        compiler_params=_CP,
        scratch_types=[
            pltpu.VMEM((N_CHUNKS, CHUNK), jnp.int32),
            pltpu.VMEM((N_PAD,), jnp.float32),
        ],
    )
    def k(dst_hbm, out_hbm, idx_v, hist_v):
        c = lax.axis_index("c")
        s = lax.axis_index("s")
        wid = s * 2 + c
        pltpu.sync_copy(dst_hbm.at[wid], idx_v)

        @pl.loop(0, N_PAD // 16)
        def _(i):
            hist_v[pl.ds(i * 16, 16)] = jnp.zeros((16,), jnp.float32)

        ones = jnp.ones((16,), jnp.float32)

        @pl.loop(0, N_CHUNKS)
        def _(j):
            @pl.loop(0, CHUNK // 16)
            def _(t):
                plsc.addupdate_scatter(
                    hist_v, [idx_v[j, pl.ds(t * 16, 16)]], ones
                )

        pltpu.sync_copy(hist_v, out_hbm.at[wid])

    return k(dst3)


def _tc_dinv(deg_part):
    def body(p_ref, out_ref):
        deg = jnp.sum(p_ref[...], axis=0, keepdims=True) + 1.0
        out_ref[...] = lax.rsqrt(deg)

    return pl.pallas_call(
        body,
        out_shape=jax.ShapeDtypeStruct((1, N_PAD), jnp.float32),
    )(deg_part)


def _sc_segsum(g, src3, dst3):
    @functools.partial(
        pl.kernel,
        out_type=jax.ShapeDtypeStruct((2, N_PAD, D), jnp.float32),
        mesh=_MESH,
        scratch_types=[
            pltpu.VMEM((2, NG, CHUNK), jnp.int32),
            pltpu.VMEM((2, NG, CHUNK), jnp.int32),
            pltpu.VMEM((2 * CHUNK, D), jnp.float32),
            pltpu.VMEM_SHARED((N_PAD, D), jnp.float32),
            pltpu.SemaphoreType.DMA,
            pltpu.SemaphoreType.DMA,
            pltpu.SemaphoreType.DMA,
            pltpu.SemaphoreType.DMA,
        ],
    )
    def k(g_hbm, src_hbm, dst_hbm, out_hbm, si, di, buf, agg_sh,
          sg0, sg1, is0, is1):
        c = lax.axis_index("c")
        s = lax.axis_index("s")
        wid = s * 2 + c
        rows = pl.ds(s * SUB_ROWS, SUB_ROWS)
        halves = (buf.at[pl.ds(0, CHUNK)], buf.at[pl.ds(CHUNK, CHUNK)])
        sgs = (sg0, sg1)
        iss = (is0, is1)

        def idx_start(q, gg):
            pltpu.async_copy(src_hbm.at[wid, pl.ds(gg * NG, NG)], si.at[q], iss[q])
            pltpu.async_copy(dst_hbm.at[wid, pl.ds(gg * NG, NG)], di.at[q], iss[q])

        def idx_wait(q, gg):
            pltpu.make_async_copy(
                src_hbm.at[wid, pl.ds(gg * NG, NG)], si.at[q], iss[q]
            ).wait()
            pltpu.make_async_copy(
                dst_hbm.at[wid, pl.ds(gg * NG, NG)], di.at[q], iss[q]
            ).wait()

        for q in range(2):
            idx_start(q, q)
        pltpu.sync_copy(g_hbm.at[rows], agg_sh.at[rows])
        plsc.subcore_barrier()

        @pl.loop(0, NGROUPS, step=2)
        def _(g):
            for q in range(2):
                gg = g + q
                idx_wait(q, gg)
                for b in range(2):
                    pltpu.async_copy(g_hbm.at[si.at[q, b]], halves[b], sgs[b])
                for t in range(NG):
                    b = t % 2
                    pltpu.make_async_copy(
                        g_hbm.at[si.at[q, t]], halves[b], sgs[b]
                    ).wait()
                    pltpu.sync_copy(halves[b], agg_sh.at[di.at[q, t]], add=True)
                    if t + 2 < NG:
                        pltpu.async_copy(g_hbm.at[si.at[q, t + 2]], halves[b], sgs[b])

                @pl.when(gg + 2 < NGROUPS)
                def _(q=q, gg=gg):
                    idx_start(q, gg + 2)

        plsc.subcore_barrier()
        pltpu.sync_copy(agg_sh.at[rows], out_hbm.at[c].at[rows])

    return k(g, src3, dst3)


def _tc_prep(dinv, x_pad, W1):
    def body(dinv_ref, x_ref, w_ref, g_ref):
        h = jnp.dot(x_ref[...], w_ref[...], preferred_element_type=jnp.float32)
        g_ref[...] = h * dinv_ref[...]

    return pl.pallas_call(
        body,
        grid=(N_PAD // BLK,),
        in_specs=[
            pl.BlockSpec((BLK, 1), lambda i: (i, 0)),
            pl.BlockSpec((BLK, D), lambda i: (i, 0)),
            pl.BlockSpec((D, D), lambda i: (0, 0)),
        ],
        out_specs=pl.BlockSpec((BLK, D), lambda i: (i, 0)),
        out_shape=jax.ShapeDtypeStruct((N_PAD, D), jnp.float32),
    )(dinv, x_pad, W1)


def _tc_layer(p, g, dinv, b2d, W_next):
    def body(p_ref, g_ref, dinv_ref, b_ref, w_ref, out_ref):
        t = (p_ref[0] + p_ref[1] - g_ref[...]) * dinv_ref[...] + b_ref[...]
        t = jnp.maximum(t, 0.0)
        out_ref[...] = (
            jnp.dot(t, w_ref[...], preferred_element_type=jnp.float32)
            * dinv_ref[...]
        )

    return pl.pallas_call(
        body,
        grid=(N_PAD // BLK,),
        in_specs=[
            pl.BlockSpec((2, BLK, D), lambda i: (0, i, 0)),
            pl.BlockSpec((BLK, D), lambda i: (i, 0)),
            pl.BlockSpec((BLK, 1), lambda i: (i, 0)),
            pl.BlockSpec((1, D), lambda i: (0, 0)),
            pl.BlockSpec((D, D), lambda i: (0, 0)),
        ],
        out_specs=pl.BlockSpec((BLK, D), lambda i: (i, 0)),
        out_shape=jax.ShapeDtypeStruct((N_PAD, D), jnp.float32),
    )(p, g, dinv, b2d, W_next)


def _tc_final(p, g, dinv, b2d):
    def body(p_ref, g_ref, dinv_ref, b_ref, out_ref):
        out_ref[...] = (
            (p_ref[0] + p_ref[1] - g_ref[...]) * dinv_ref[...] + b_ref[...]
        )

    return pl.pallas_call(
        body,
        grid=(N_PAD // BLK,),
        in_specs=[
            pl.BlockSpec((2, BLK, D), lambda i: (0, i, 0)),
            pl.BlockSpec((BLK, D), lambda i: (i, 0)),
            pl.BlockSpec((BLK, 1), lambda i: (i, 0)),
            pl.BlockSpec((1, D), lambda i: (0, 0)),
        ],
        out_specs=pl.BlockSpec((BLK, D), lambda i: (i, 0)),
        out_shape=jax.ShapeDtypeStruct((N_PAD, D), jnp.float32),
    )(p, g, dinv, b2d)


def kernel(x, edge_index, W1, b1, W2, b2, W3, b3):
    src = edge_index[0]
    dst = edge_index[1]
    pad_e = E_PAD - E
    pad_iota = jnp.arange(pad_e, dtype=jnp.int32)
    src3 = jnp.concatenate(
        [src, pad_iota % N]
    ).reshape(NW, N_CHUNKS, CHUNK)
    dst3 = jnp.concatenate(
        [dst, DUMP_ROW + pad_iota % (N_PAD - N)]
    ).reshape(NW, N_CHUNKS, CHUNK)
    x_pad = jnp.pad(x, ((0, N_PAD - N), (0, 0)))

    deg_part = _sc_degree(dst3)
    dinv = _tc_dinv(deg_part).reshape(N_PAD, 1)
    g1 = _tc_prep(dinv, x_pad, W1)
    p1 = _sc_segsum(g1, src3, dst3)
    g2 = _tc_layer(p1, g1, dinv, b1.reshape(1, D), W2)
    p2 = _sc_segsum(g2, src3, dst3)
    g3 = _tc_layer(p2, g2, dinv, b2.reshape(1, D), W3)
    p3 = _sc_segsum(g3, src3, dst3)
    return _tc_final(p3, g3, dinv, b3.reshape(1, D))[:N]


# speedup vs baseline: 24.9278x; 1.0500x over previous
import dataclasses
import functools

import jax
import jax.numpy as jnp
from jax import lax
from jax.experimental import pallas as pl
from jax.experimental.pallas import tpu as pltpu
from jax.experimental.pallas import tpu_sc as plsc

N = 10000
D = 128
E = 320000

N_PAD = 10240
E_PAD = 327680
NW = 32
EDGES_PER_TILE = E_PAD // NW
CHUNK = 64
N_CHUNKS = EDGES_PER_TILE // CHUNK
NG = 16
NGROUPS = N_CHUNKS // NG
SUB_ROWS = N_PAD // 16
DUMP_ROW = N
BLK = 1024

_MESH = plsc.VectorSubcoreMesh(core_axis_name="c", subcore_axis_name="s")

_CP = pltpu.CompilerParams()
if "needs_layout_passes" in pltpu.CompilerParams.__dataclass_fields__:
    _CP = dataclasses.replace(_CP, needs_layout_passes=False)


def _sc_degree(dst3):
    @functools.partial(
        pl.kernel,
        out_type=jax.ShapeDtypeStruct((NW, N_PAD), jnp.float32),
        mesh=_MESH,
        compiler_params=_CP,
        scratch_types=[
            pltpu.VMEM((N_CHUNKS, CHUNK), jnp.int32),
            pltpu.VMEM((N_PAD,), jnp.float32),
        ],
    )
    def k(dst_hbm, out_hbm, idx_v, hist_v):
        c = lax.axis_index("c")
        s = lax.axis_index("s")
        wid = s * 2 + c
        pltpu.sync_copy(dst_hbm.at[wid], idx_v)

        @pl.loop(0, N_PAD // 16)
        def _(i):
            hist_v[pl.ds(i * 16, 16)] = jnp.zeros((16,), jnp.float32)

        ones = jnp.ones((16,), jnp.float32)

        @pl.loop(0, N_CHUNKS)
        def _(j):
            @pl.loop(0, CHUNK // 16)
            def _(t):
                plsc.addupdate_scatter(
                    hist_v, [idx_v[j, pl.ds(t * 16, 16)]], ones
                )

        pltpu.sync_copy(hist_v, out_hbm.at[wid])

    return k(dst3)


def _tc_dinv(deg_part):
    def body(p_ref, out_ref):
        deg = jnp.sum(p_ref[...], axis=0, keepdims=True) + 1.0
        out_ref[...] = lax.rsqrt(deg)

    return pl.pallas_call(
        body,
        out_shape=jax.ShapeDtypeStruct((1, N_PAD), jnp.float32),
    )(deg_part)


def _sc_segsum(g, src3, dst3):
    @functools.partial(
        pl.kernel,
        out_type=jax.ShapeDtypeStruct((2, N_PAD, D), jnp.float32),
        mesh=_MESH,
        scratch_types=[
            pltpu.VMEM((2, NG, CHUNK), jnp.int32),
            pltpu.VMEM((2, NG, CHUNK), jnp.int32),
            pltpu.VMEM((2 * CHUNK, D), jnp.float32),
            pltpu.VMEM_SHARED((N_PAD, D), jnp.float32),
            pltpu.SemaphoreType.DMA,
            pltpu.SemaphoreType.DMA,
            pltpu.SemaphoreType.DMA,
            pltpu.SemaphoreType.DMA,
        ],
    )
    def k(g_hbm, src_hbm, dst_hbm, out_hbm, si, di, buf, agg_sh,
          sg0, sg1, is0, is1):
        c = lax.axis_index("c")
        s = lax.axis_index("s")
        wid = s * 2 + c
        rows = pl.ds(s * SUB_ROWS, SUB_ROWS)
        halves = (buf.at[pl.ds(0, CHUNK)], buf.at[pl.ds(CHUNK, CHUNK)])
        sgs = (sg0, sg1)
        iss = (is0, is1)

        def idx_start(q, gg):
            pltpu.async_copy(src_hbm.at[wid, pl.ds(gg * NG, NG)], si.at[q], iss[q])
            pltpu.async_copy(dst_hbm.at[wid, pl.ds(gg * NG, NG)], di.at[q], iss[q])

        def idx_wait(q, gg):
            pltpu.make_async_copy(
                src_hbm.at[wid, pl.ds(gg * NG, NG)], si.at[q], iss[q]
            ).wait()
            pltpu.make_async_copy(
                dst_hbm.at[wid, pl.ds(gg * NG, NG)], di.at[q], iss[q]
            ).wait()

        idx_start(0, 0)
        idx_start(1, 1)
        pltpu.sync_copy(g_hbm.at[rows], agg_sh.at[rows])
        idx_wait(0, 0)
        for b in range(2):
            pltpu.async_copy(g_hbm.at[si.at[0, b]], halves[b], sgs[b])
        plsc.subcore_barrier()

        @pl.loop(0, NGROUPS, step=2)
        def _(g):
            for q in range(2):
                gg = g + q
                qn = 1 - q
                for t in range(NG - 2):
                    b = t % 2
                    pltpu.make_async_copy(
                        g_hbm.at[si.at[q, t]], halves[b], sgs[b]
                    ).wait()
                    pltpu.sync_copy(halves[b], agg_sh.at[di.at[q, t]], add=True)
                    pltpu.async_copy(g_hbm.at[si.at[q, t + 2]], halves[b], sgs[b])

                @pl.when(gg + 1 < NGROUPS)
                def _(q=q, qn=qn, gg=gg):
                    idx_wait(qn, gg + 1)
                    for t in range(NG - 2, NG):
                        b = t % 2
                        pltpu.make_async_copy(
                            g_hbm.at[si.at[q, t]], halves[b], sgs[b]
                        ).wait()
                        pltpu.sync_copy(halves[b], agg_sh.at[di.at[q, t]], add=True)
                        pltpu.async_copy(
                            g_hbm.at[si.at[qn, t - (NG - 2)]], halves[b], sgs[b]
                        )

                @pl.when(gg + 1 >= NGROUPS)
                def _(q=q, gg=gg):
                    for t in range(NG - 2, NG):
                        b = t % 2
                        pltpu.make_async_copy(
                            g_hbm.at[si.at[q, t]], halves[b], sgs[b]
                        ).wait()
                        pltpu.sync_copy(halves[b], agg_sh.at[di.at[q, t]], add=True)

                @pl.when(gg + 2 < NGROUPS)
                def _(q=q, gg=gg):
                    idx_start(q, gg + 2)

        plsc.subcore_barrier()
        pltpu.sync_copy(agg_sh.at[rows], out_hbm.at[c].at[rows])

    return k(g, src3, dst3)


def _tc_prep(dinv, x_pad, W1):
    def body(dinv_ref, x_ref, w_ref, g_ref):
        h = jnp.dot(x_ref[...], w_ref[...], preferred_element_type=jnp.float32)
        g_ref[...] = h * dinv_ref[...]

    return pl.pallas_call(
        body,
        grid=(N_PAD // BLK,),
        in_specs=[
            pl.BlockSpec((BLK, 1), lambda i: (i, 0)),
            pl.BlockSpec((BLK, D), lambda i: (i, 0)),
            pl.BlockSpec((D, D), lambda i: (0, 0)),
        ],
        out_specs=pl.BlockSpec((BLK, D), lambda i: (i, 0)),
        out_shape=jax.ShapeDtypeStruct((N_PAD, D), jnp.float32),
    )(dinv, x_pad, W1)


def _tc_layer(p, g, dinv, b2d, W_next):
    def body(p_ref, g_ref, dinv_ref, b_ref, w_ref, out_ref):
        t = (p_ref[0] + p_ref[1] - g_ref[...]) * dinv_ref[...] + b_ref[...]
        t = jnp.maximum(t, 0.0)
        out_ref[...] = (
            jnp.dot(t, w_ref[...], preferred_element_type=jnp.float32)
            * dinv_ref[...]
        )

    return pl.pallas_call(
        body,
        grid=(N_PAD // BLK,),
        in_specs=[
            pl.BlockSpec((2, BLK, D), lambda i: (0, i, 0)),
            pl.BlockSpec((BLK, D), lambda i: (i, 0)),
            pl.BlockSpec((BLK, 1), lambda i: (i, 0)),
            pl.BlockSpec((1, D), lambda i: (0, 0)),
            pl.BlockSpec((D, D), lambda i: (0, 0)),
        ],
        out_specs=pl.BlockSpec((BLK, D), lambda i: (i, 0)),
        out_shape=jax.ShapeDtypeStruct((N_PAD, D), jnp.float32),
    )(p, g, dinv, b2d, W_next)


def _tc_final(p, g, dinv, b2d):
    def body(p_ref, g_ref, dinv_ref, b_ref, out_ref):
        out_ref[...] = (
            (p_ref[0] + p_ref[1] - g_ref[...]) * dinv_ref[...] + b_ref[...]
        )

    return pl.pallas_call(
        body,
        grid=(N_PAD // BLK,),
        in_specs=[
            pl.BlockSpec((2, BLK, D), lambda i: (0, i, 0)),
            pl.BlockSpec((BLK, D), lambda i: (i, 0)),
            pl.BlockSpec((BLK, 1), lambda i: (i, 0)),
            pl.BlockSpec((1, D), lambda i: (0, 0)),
        ],
        out_specs=pl.BlockSpec((BLK, D), lambda i: (i, 0)),
        out_shape=jax.ShapeDtypeStruct((N_PAD, D), jnp.float32),
    )(p, g, dinv, b2d)


def kernel(x, edge_index, W1, b1, W2, b2, W3, b3):
    src = edge_index[0]
    dst = edge_index[1]
    pad_e = E_PAD - E
    pad_iota = jnp.arange(pad_e, dtype=jnp.int32)
    src3 = jnp.concatenate(
        [src, pad_iota % N]
    ).reshape(NW, N_CHUNKS, CHUNK)
    dst3 = jnp.concatenate(
        [dst, DUMP_ROW + pad_iota % (N_PAD - N)]
    ).reshape(NW, N_CHUNKS, CHUNK)
    x_pad = jnp.pad(x, ((0, N_PAD - N), (0, 0)))

    deg_part = _sc_degree(dst3)
    dinv = _tc_dinv(deg_part).reshape(N_PAD, 1)
    g1 = _tc_prep(dinv, x_pad, W1)
    p1 = _sc_segsum(g1, src3, dst3)
    g2 = _tc_layer(p1, g1, dinv, b1.reshape(1, D), W2)
    p2 = _sc_segsum(g2, src3, dst3)
    g3 = _tc_layer(p2, g2, dinv, b2.reshape(1, D), W3)
    p3 = _sc_segsum(g3, src3, dst3)
    return _tc_final(p3, g3, dinv, b3.reshape(1, D))[:N]
